# Initial kernel scaffold; baseline (speedup 1.0000x reference)
import jax
import jax.numpy as jnp
from jax.experimental import pallas as pl


def kernel(center_words, target_words, outer_words, emb_v, emb_u):
    raise NotImplementedError("write your pallas kernel here")


# speedup vs baseline: 4.1692x; 4.1692x over previous
import functools

import jax
import jax.numpy as jnp
from jax import lax
from jax.experimental import pallas as pl
from jax.experimental.pallas import tpu as pltpu
from jax.experimental.pallas import tpu_sc as plsc

D = 64
K = 20
CH = 16


def _sc_body(cw_hbm, tw_hbm, ow_hbm, emb_v_hbm, emb_u_hbm,
             scores_hbm, sumexp_hbm,
             idxc_v, idxt_v, idxo_v, rows_c, rows_t, rows_o,
             scores_v, sumexp_v, sem0, sem1, sem2,
             *, bpw, nchunk):
  nc = plsc.get_sparse_core_info().num_cores
  wid = lax.axis_index("s") * nc + lax.axis_index("c")
  rowids = lax.iota(jnp.int32, CH)

  def chunk_body(c, carry):
    base = wid * bpw + c * CH
    pltpu.sync_copy(cw_hbm.at[pl.ds(base, CH)], idxc_v)
    pltpu.sync_copy(tw_hbm.at[pl.ds(base, CH)], idxt_v)
    pltpu.sync_copy(ow_hbm.at[pl.ds(base * K, CH * K)], idxo_v)
    cp0 = pltpu.async_copy(emb_v_hbm.at[idxc_v], rows_c, sem0)
    cp1 = pltpu.async_copy(emb_u_hbm.at[idxt_v], rows_t, sem1)
    cp2 = pltpu.async_copy(emb_u_hbm.at[idxo_v], rows_o, sem2)
    cp0.wait()
    cp1.wait()
    cp2.wait()

    score = jnp.zeros((CH,), jnp.float32)
    for d in range(D):
      col = rowids * 0 + d
      cv = plsc.load_gather(rows_c, [rowids, col])
      tv = plsc.load_gather(rows_t, [rowids, col])
      score = score + cv * tv

    def k_body(k, se):
      orow = rowids * K + k
      norm = jnp.zeros((CH,), jnp.float32)
      for d in range(D):
        col = rowids * 0 + d
        cv = plsc.load_gather(rows_c, [rowids, col])
        ov = plsc.load_gather(rows_o, [orow, col])
        norm = norm + cv * ov
      return se + jnp.exp(norm)

    sumexp = lax.fori_loop(0, K, k_body, jnp.zeros((CH,), jnp.float32))

    scores_v[pl.ds(c * CH, CH)] = score
    sumexp_v[pl.ds(c * CH, CH)] = sumexp
    return carry

  lax.fori_loop(0, nchunk, chunk_body, 0)
  pltpu.sync_copy(scores_v, scores_hbm.at[pl.ds(wid * bpw, bpw)])
  pltpu.sync_copy(sumexp_v, sumexp_hbm.at[pl.ds(wid * bpw, bpw)])


def _sc_gather_dots(cw, tw, ow, emb_v, emb_u):
  b = cw.shape[0]
  info = plsc.get_sparse_core_info()
  nw = info.num_cores * info.num_subcores
  bpw = b // nw
  nchunk = bpw // CH
  mesh = plsc.VectorSubcoreMesh(core_axis_name="c", subcore_axis_name="s")
  f32 = jnp.float32
  run = pl.kernel(
      functools.partial(_sc_body, bpw=bpw, nchunk=nchunk),
      out_type=(jax.ShapeDtypeStruct((b,), f32),
                jax.ShapeDtypeStruct((b,), f32)),
      mesh=mesh,
      compiler_params=pltpu.CompilerParams(needs_layout_passes=False,
                                           use_tc_tiling_on_sc=False),
      scratch_types=[
          pltpu.VMEM((CH,), jnp.int32),
          pltpu.VMEM((CH,), jnp.int32),
          pltpu.VMEM((CH * K,), jnp.int32),
          pltpu.VMEM((CH, D), f32),
          pltpu.VMEM((CH, D), f32),
          pltpu.VMEM((CH * K, D), f32),
          pltpu.VMEM((bpw,), f32),
          pltpu.VMEM((bpw,), f32),
          pltpu.SemaphoreType.DMA,
          pltpu.SemaphoreType.DMA,
          pltpu.SemaphoreType.DMA,
      ],
  )
  return run(cw, tw, ow, emb_v, emb_u)


def _finish_body(s_ref, e_ref, o_ref):
  s = s_ref[...]
  e = e_ref[...]
  n = s.size
  o_ref[0, 0] = -(jnp.sum(s) - jnp.sum(jnp.log(e))) / n


def _tc_finish(scores, sumexp):
  b = scores.shape[0]
  rows = b // 128
  out = pl.pallas_call(
      _finish_body,
      out_shape=jax.ShapeDtypeStruct((1, 1), jnp.float32),
      out_specs=pl.BlockSpec(memory_space=pltpu.SMEM),
  )(scores.reshape(rows, 128), sumexp.reshape(rows, 128))
  return out[0, 0]


def kernel(center_words, target_words, outer_words, emb_v, emb_u):
  cw = center_words.reshape(-1).astype(jnp.int32)
  tw = target_words.reshape(-1).astype(jnp.int32)
  ow = outer_words.reshape(-1).astype(jnp.int32)
  scores, sumexp = _sc_gather_dots(cw, tw, ow, emb_v, emb_u)
  return _tc_finish(scores, sumexp)


# speedup vs baseline: 4.3513x; 1.0437x over previous
import functools

import jax
import jax.numpy as jnp
from jax import lax
from jax.experimental import pallas as pl
from jax.experimental.pallas import tpu as pltpu
from jax.experimental.pallas import tpu_sc as plsc

D = 64
K = 20
CH = 16
HALF = D // 2


def _sc_body(cw_hbm, tw_hbm, ow_hbm, emb_v_hbm, emb_u_hbm,
             scores_hbm, sumexp_hbm,
             idxc, idxt, idxo, rows_c, rows_t, rows_o,
             nscr, scores_v, sumexp_v, semc, semt, semo,
             *, bpw, nchunk):
  nc = plsc.get_sparse_core_info().num_cores
  wid = lax.axis_index("s") * nc + lax.axis_index("c")
  rowids = lax.iota(jnp.int32, CH)

  def issue(chunk, p):
    base = wid * bpw + chunk * CH
    pltpu.sync_copy(cw_hbm.at[pl.ds(base, CH)], idxc[p])
    pltpu.sync_copy(tw_hbm.at[pl.ds(base, CH)], idxt[p])
    pltpu.sync_copy(ow_hbm.at[pl.ds(base * K, CH * K)], idxo[p])
    pltpu.async_copy(emb_v_hbm.at[idxc[p]], rows_c[p], semc[p])
    pltpu.async_copy(emb_u_hbm.at[idxt[p]], rows_t[p], semt[p])
    pltpu.async_copy(emb_u_hbm.at[idxo[p]], rows_o[p], semo[p])

  def drain(p):
    pltpu.make_async_copy(emb_v_hbm.at[pl.ds(0, CH)], rows_c[p], semc[p]).wait()
    pltpu.make_async_copy(emb_u_hbm.at[pl.ds(0, CH)], rows_t[p], semt[p]).wait()
    pltpu.make_async_copy(emb_u_hbm.at[pl.ds(0, CH * K)], rows_o[p],
                          semo[p]).wait()

  issue(0, 0)

  def pair_body(pair, carry):
    for p in (0, 1):
      c = 2 * pair + p
      drain(p)
      issue(jnp.minimum(c + 1, nchunk - 1), 1 - p)

      score = jnp.zeros((CH,), jnp.float32)
      sumexp = jnp.zeros((CH,), jnp.float32)
      for half in (0, 1):
        base_d = half * HALF
        ccols = [plsc.load_gather(rows_c[p], [rowids, rowids * 0 + base_d + d])
                 for d in range(HALF)]
        tacc = jnp.zeros((CH,), jnp.float32)
        for d in range(HALF):
          tv = plsc.load_gather(rows_t[p], [rowids, rowids * 0 + base_d + d])
          tacc = tacc + ccols[d] * tv
        score = score + tacc

        def k_body(k, se, *, _p=p, _ccols=ccols, _base_d=base_d, _half=half):
          orow = rowids * K + k
          nacc = jnp.zeros((CH,), jnp.float32)
          for d in range(HALF):
            ov = plsc.load_gather(rows_o[_p],
                                  [orow, rowids * 0 + _base_d + d])
            nacc = nacc + _ccols[d] * ov
          if _half == 0:
            nscr[pl.ds(k * CH, CH)] = nacc
            return se
          return se + jnp.exp(nscr[pl.ds(k * CH, CH)] + nacc)

        sumexp = lax.fori_loop(0, K, k_body, sumexp)

      scores_v[pl.ds(c * CH, CH)] = score
      sumexp_v[pl.ds(c * CH, CH)] = sumexp
    return carry

  lax.fori_loop(0, nchunk // 2, pair_body, 0)
  drain(0)
  pltpu.sync_copy(scores_v, scores_hbm.at[pl.ds(wid * bpw, bpw)])
  pltpu.sync_copy(sumexp_v, sumexp_hbm.at[pl.ds(wid * bpw, bpw)])


def _sc_gather_dots(cw, tw, ow, emb_v, emb_u):
  b = cw.shape[0]
  info = plsc.get_sparse_core_info()
  nw = info.num_cores * info.num_subcores
  bpw = b // nw
  nchunk = bpw // CH
  mesh = plsc.VectorSubcoreMesh(core_axis_name="c", subcore_axis_name="s")
  f32 = jnp.float32
  i32 = jnp.int32
  run = pl.kernel(
      functools.partial(_sc_body, bpw=bpw, nchunk=nchunk),
      out_type=(jax.ShapeDtypeStruct((b,), f32),
                jax.ShapeDtypeStruct((b,), f32)),
      mesh=mesh,
      compiler_params=pltpu.CompilerParams(needs_layout_passes=False,
                                           use_tc_tiling_on_sc=False),
      scratch_types=[
          [pltpu.VMEM((CH,), i32)] * 2,
          [pltpu.VMEM((CH,), i32)] * 2,
          [pltpu.VMEM((CH * K,), i32)] * 2,
          [pltpu.VMEM((CH, D), f32)] * 2,
          [pltpu.VMEM((CH, D), f32)] * 2,
          [pltpu.VMEM((CH * K, D), f32)] * 2,
          pltpu.VMEM((CH * K,), f32),
          pltpu.VMEM((bpw,), f32),
          pltpu.VMEM((bpw,), f32),
          [pltpu.SemaphoreType.DMA] * 2,
          [pltpu.SemaphoreType.DMA] * 2,
          [pltpu.SemaphoreType.DMA] * 2,
      ],
  )
  return run(cw, tw, ow, emb_v, emb_u)


def _finish_body(s_ref, e_ref, o_ref):
  s = s_ref[...]
  e = e_ref[...]
  n = s.size
  o_ref[0, 0] = -(jnp.sum(s) - jnp.sum(jnp.log(e))) / n


def _tc_finish(scores, sumexp):
  b = scores.shape[0]
  rows = b // 128
  out = pl.pallas_call(
      _finish_body,
      out_shape=jax.ShapeDtypeStruct((1, 1), jnp.float32),
      out_specs=pl.BlockSpec(memory_space=pltpu.SMEM),
  )(scores.reshape(rows, 128), sumexp.reshape(rows, 128))
  return out[0, 0]


def kernel(center_words, target_words, outer_words, emb_v, emb_u):
  cw = center_words.reshape(-1).astype(jnp.int32)
  tw = target_words.reshape(-1).astype(jnp.int32)
  ow = outer_words.reshape(-1).astype(jnp.int32)
  scores, sumexp = _sc_gather_dots(cw, tw, ow, emb_v, emb_u)
  return _tc_finish(scores, sumexp)


# speedup vs baseline: 5.1705x; 1.1883x over previous
import functools

import jax
import jax.numpy as jnp
from jax import lax
from jax.experimental import pallas as pl
from jax.experimental.pallas import tpu as pltpu
from jax.experimental.pallas import tpu_sc as plsc

D = 64
K = 20
CH = 16
HALF = D // 2


def _sc_body(cw_hbm, tw_hbm, ow_hbm, emb_v_hbm, emb_u_hbm,
             scores_hbm, sumexp_hbm,
             idxc, idxt, idxo, rows_c, rows_t, rows_o,
             nscr, scores_v, sumexp_v, semc, semt, semo,
             *, bpw, nchunk):
  nc = plsc.get_sparse_core_info().num_cores
  wid = lax.axis_index("s") * nc + lax.axis_index("c")
  rowids = lax.iota(jnp.int32, CH)

  base0 = wid * bpw
  pltpu.sync_copy(cw_hbm.at[pl.ds(base0, bpw)], idxc)
  pltpu.sync_copy(tw_hbm.at[pl.ds(base0, bpw)], idxt)
  pltpu.sync_copy(ow_hbm.at[pl.ds(base0 * K, bpw * K)], idxo)

  def issue(chunk, p):
    pltpu.async_copy(emb_v_hbm.at[idxc.at[pl.ds(chunk * CH, CH)]],
                     rows_c[p], semc[p])
    pltpu.async_copy(emb_u_hbm.at[idxt.at[pl.ds(chunk * CH, CH)]],
                     rows_t[p], semt[p])
    pltpu.async_copy(emb_u_hbm.at[idxo.at[pl.ds(chunk * CH * K, CH * K)]],
                     rows_o[p], semo[p])

  def drain(p):
    pltpu.make_async_copy(emb_v_hbm.at[pl.ds(0, CH)], rows_c[p], semc[p]).wait()
    pltpu.make_async_copy(emb_u_hbm.at[pl.ds(0, CH)], rows_t[p], semt[p]).wait()
    pltpu.make_async_copy(emb_u_hbm.at[pl.ds(0, CH * K)], rows_o[p],
                          semo[p]).wait()

  issue(0, 0)

  def pair_body(pair, carry):
    for p in (0, 1):
      c = 2 * pair + p
      drain(p)
      issue(jnp.minimum(c + 1, nchunk - 1), 1 - p)

      score = jnp.zeros((CH,), jnp.float32)
      sumexp = jnp.zeros((CH,), jnp.float32)
      for half in (0, 1):
        base_d = half * HALF
        cols = [(rowids + base_d + d) & (D - 1) for d in range(HALF)]
        ccols = [plsc.load_gather(rows_c[p], [rowids, cols[d]])
                 for d in range(HALF)]
        tacc = [jnp.zeros((CH,), jnp.float32) for _ in range(4)]
        for d in range(HALF):
          tv = plsc.load_gather(rows_t[p], [rowids, cols[d]])
          tacc[d % 4] = tacc[d % 4] + ccols[d] * tv
        score = score + (tacc[0] + tacc[1]) + (tacc[2] + tacc[3])

        def k_body(k, se, *, _p=p, _ccols=ccols, _cols=cols, _half=half):
          orow = rowids * K + k
          nacc = [jnp.zeros((CH,), jnp.float32) for _ in range(4)]
          for d in range(HALF):
            ov = plsc.load_gather(rows_o[_p], [orow, _cols[d]])
            nacc[d % 4] = nacc[d % 4] + _ccols[d] * ov
          total = (nacc[0] + nacc[1]) + (nacc[2] + nacc[3])
          if _half == 0:
            nscr[pl.ds(k * CH, CH)] = total
            return se
          return se + jnp.exp(nscr[pl.ds(k * CH, CH)] + total)

        sumexp = lax.fori_loop(0, K, k_body, sumexp)

      scores_v[pl.ds(c * CH, CH)] = score
      sumexp_v[pl.ds(c * CH, CH)] = sumexp
    return carry

  lax.fori_loop(0, nchunk // 2, pair_body, 0)
  drain(0)
  pltpu.sync_copy(scores_v, scores_hbm.at[pl.ds(wid * bpw, bpw)])
  pltpu.sync_copy(sumexp_v, sumexp_hbm.at[pl.ds(wid * bpw, bpw)])


def _sc_gather_dots(cw, tw, ow, emb_v, emb_u):
  b = cw.shape[0]
  info = plsc.get_sparse_core_info()
  nw = info.num_cores * info.num_subcores
  bpw = b // nw
  nchunk = bpw // CH
  mesh = plsc.VectorSubcoreMesh(core_axis_name="c", subcore_axis_name="s")
  f32 = jnp.float32
  i32 = jnp.int32
  run = pl.kernel(
      functools.partial(_sc_body, bpw=bpw, nchunk=nchunk),
      out_type=(jax.ShapeDtypeStruct((b,), f32),
                jax.ShapeDtypeStruct((b,), f32)),
      mesh=mesh,
      compiler_params=pltpu.CompilerParams(needs_layout_passes=False,
                                           use_tc_tiling_on_sc=False),
      scratch_types=[
          pltpu.VMEM((bpw,), i32),
          pltpu.VMEM((bpw,), i32),
          pltpu.VMEM((bpw * K,), i32),
          [pltpu.VMEM((CH, D), f32)] * 2,
          [pltpu.VMEM((CH, D), f32)] * 2,
          [pltpu.VMEM((CH * K, D), f32)] * 2,
          pltpu.VMEM((CH * K,), f32),
          pltpu.VMEM((bpw,), f32),
          pltpu.VMEM((bpw,), f32),
          [pltpu.SemaphoreType.DMA] * 2,
          [pltpu.SemaphoreType.DMA] * 2,
          [pltpu.SemaphoreType.DMA] * 2,
      ],
  )
  return run(cw, tw, ow, emb_v, emb_u)


def _finish_body(s_ref, e_ref, o_ref):
  s = s_ref[...]
  e = e_ref[...]
  n = s.size
  o_ref[0, 0] = -(jnp.sum(s) - jnp.sum(jnp.log(e))) / n


def _tc_finish(scores, sumexp):
  b = scores.shape[0]
  rows = b // 128
  out = pl.pallas_call(
      _finish_body,
      out_shape=jax.ShapeDtypeStruct((1, 1), jnp.float32),
      out_specs=pl.BlockSpec(memory_space=pltpu.SMEM),
  )(scores.reshape(rows, 128), sumexp.reshape(rows, 128))
  return out[0, 0]


def kernel(center_words, target_words, outer_words, emb_v, emb_u):
  cw = center_words.reshape(-1).astype(jnp.int32)
  tw = target_words.reshape(-1).astype(jnp.int32)
  ow = outer_words.reshape(-1).astype(jnp.int32)
  scores, sumexp = _sc_gather_dots(cw, tw, ow, emb_v, emb_u)
  return _tc_finish(scores, sumexp)
